# baseline (device time: 178235 ns/iter reference)
import jax
import jax.numpy as jnp
from jax import lax
from jax.experimental import pallas as pl
from jax.experimental.pallas import tpu as pltpu

N_DEV = 8
N_SYS = 3
N_STEPS = 7

E = [
    [1, 2, 1, 4, 1, 2, 1],
    [2, 4, 2, 1, 2, 4, 2],
    [4, 1, 4, 2, 4, 1, 4],
]
MX = [
    [4, 5, 7, 6, 2, 3, 1, 0],
    [1, 3, 7, 5, 4, 6, 2, 0],
    [2, 6, 7, 3, 1, 5, 4, 0],
]
COLS = [(0, 1408), (1408, 1408), (2816, 1280)]
STREAMS = [
    (0, 0, 704), (1, 1408, 704), (2, 2816, 640),
    (0, 704, 704), (1, 2112, 704), (2, 3456, 640),
]
NCMAX = 704


def kernel(A, B):
    m, k_per = A.shape
    _, n = B.shape
    m_per = m // N_DEV

    def body(a_hbm, b_hbm, out_ref, b16, bstage, astage, comm, pbuf,
             dma_sems, send_sems, recv_sems, ready_sems):
        d = lax.axis_index("i")

        def to_vertex(q):
            return q ^ ((q >> 1) & 1)

        v = to_vertex(d)

        def nbr(sys, t):
            return to_vertex(v ^ E[sys][t])

        def chunk_row(sys, t):
            return to_vertex(v ^ MX[sys][t]) * m_per

        def a_chunk_dma(sys, t):
            return pltpu.make_async_copy(
                a_hbm.at[pl.ds(chunk_row(sys, t), m_per), :],
                astage.at[sys],
                dma_sems.at[sys],
            )

        def dot16(sys, col0, w):
            return jnp.dot(astage[sys, :, :].astype(jnp.bfloat16),
                           b16[:, col0:col0 + w],
                           preferred_element_type=jnp.float32)

        def step_rdma(si, t, s, r):
            sys = STREAMS[si][0]
            return pltpu.make_async_remote_copy(
                src_ref=comm.at[si, s], dst_ref=comm.at[si, r],
                send_sem=send_sems.at[si, s],
                recv_sem=recv_sems.at[si, r],
                device_id=(nbr(sys, t),),
                device_id_type=pl.DeviceIdType.MESH,
            )

        barrier_sem = pltpu.get_barrier_semaphore()
        for e in (1, 2, 4):
            pl.semaphore_signal(
                barrier_sem, inc=1,
                device_id=(to_vertex(v ^ e),),
                device_id_type=pl.DeviceIdType.MESH,
            )
        pl.semaphore_wait(barrier_sem, 3)

        for sys in range(N_SYS):
            a_chunk_dma(sys, 0).start()
        for sys in range(N_SYS):
            col0, w = COLS[sys]
            for off in range(0, w, 512):
                pw = min(512, w - off)
                bdma = pltpu.make_async_copy(
                    b_hbm.at[:, pl.ds(col0 + off, pw)],
                    bstage.at[:, pl.ds(0, pw)],
                    dma_sems.at[N_SYS])
                bdma.start()
                bdma.wait()
                b16[:, col0 + off:col0 + off + pw] = (
                    bstage[:, 0:pw].astype(jnp.bfloat16))
            a_chunk_dma(sys, 0).wait()
            for si, (ssys, scol0, sw) in enumerate(STREAMS):
                if ssys == sys:
                    comm[si, 0, :, 0:sw] = dot16(
                        sys, scol0, sw).astype(jnp.bfloat16)
                    step_rdma(si, 0, 0, 1).start()
        for sys in range(N_SYS):
            a_chunk_dma(sys, 1).start()

        for t in range(N_STEPS):
            s = t % 2
            r = (t + 1) % 2
            for sys in range(N_SYS):
                pltpu.make_async_copy(
                    a_hbm.at[pl.ds(0, m_per), :], astage.at[sys],
                    dma_sems.at[sys]).wait()
            for si, (sys, col0, w) in enumerate(STREAMS):
                pbuf[si, :, 0:w] = dot16(sys, col0, w)
            if t < N_STEPS - 1:
                for sys in range(N_SYS):
                    a_chunk_dma(sys, t + 2).start()
            for si, (sys, col0, w) in enumerate(STREAMS):
                step_rdma(si, t, s, r).wait_send()
                if t <= N_STEPS - 2:
                    pl.semaphore_signal(
                        ready_sems.at[si, s], inc=1,
                        device_id=(nbr(sys, t + 1),),
                        device_id_type=pl.DeviceIdType.MESH,
                    )
            for si, (sys, col0, w) in enumerate(STREAMS):
                step_rdma(si, t, s, r).wait_recv()
                if t < N_STEPS - 1:
                    comm[si, r, :, 0:w] = (
                        comm[si, r, :, 0:w].astype(jnp.float32)
                        + pbuf[si, :, 0:w]
                    ).astype(jnp.bfloat16)
                    pl.semaphore_wait(ready_sems.at[si, s], 1)
                    step_rdma(si, t + 1, r, s).start()
                else:
                    out_ref[:, col0:col0 + w] = (
                        comm[si, r, :, 0:w].astype(jnp.float32)
                        + pbuf[si, :, 0:w]
                    )

    return pl.pallas_call(
        body,
        out_shape=jax.ShapeDtypeStruct((m_per, n), jnp.float32),
        in_specs=[
            pl.BlockSpec(memory_space=pl.ANY),
            pl.BlockSpec(memory_space=pl.ANY),
        ],
        out_specs=pl.BlockSpec(memory_space=pltpu.VMEM),
        scratch_shapes=[
            pltpu.VMEM((k_per, n), jnp.bfloat16),
            pltpu.VMEM((k_per, 512), jnp.float32),
            pltpu.VMEM((N_SYS, m_per, k_per), jnp.float32),
            pltpu.VMEM((len(STREAMS), 2, m_per, NCMAX), jnp.bfloat16),
            pltpu.VMEM((len(STREAMS), m_per, NCMAX), jnp.float32),
            pltpu.SemaphoreType.DMA((N_SYS + 1,)),
            pltpu.SemaphoreType.DMA((len(STREAMS), 2)),
            pltpu.SemaphoreType.DMA((len(STREAMS), 2)),
            pltpu.SemaphoreType.REGULAR((len(STREAMS), 2)),
        ],
        compiler_params=pltpu.CompilerParams(
            collective_id=0,
            vmem_limit_bytes=64 * 1024 * 1024,
        ),
    )(A, B)


# device time: 170544 ns/iter; 1.0451x vs baseline; 1.0451x over previous
import jax
import jax.numpy as jnp
from jax import lax
from jax.experimental import pallas as pl
from jax.experimental.pallas import tpu as pltpu

N_DEV = 8
N_SYS = 3
N_STEPS = 7

E = [
    [1, 2, 1, 4, 1, 2, 1],
    [2, 4, 2, 1, 2, 4, 2],
    [4, 1, 4, 2, 4, 1, 4],
]
MX = [
    [4, 5, 7, 6, 2, 3, 1, 0],
    [1, 3, 7, 5, 4, 6, 2, 0],
    [2, 6, 7, 3, 1, 5, 4, 0],
]
COLS = [(0, 1408), (1408, 1408), (2816, 1280)]
NCMAX = 1408


def kernel(A, B):
    m, k_per = A.shape
    _, n = B.shape
    m_per = m // N_DEV

    def body(a_hbm, b_hbm, out_ref, b16, bstage, astage, comm, pbuf,
             dma_sems, send_sems, recv_sems, ready_sems):
        d = lax.axis_index("i")

        def to_vertex(q):
            return q ^ ((q >> 1) & 1)

        v = to_vertex(d)

        def nbr(sys, t):
            return to_vertex(v ^ E[sys][t])

        def chunk_row(sys, t):
            return to_vertex(v ^ MX[sys][t]) * m_per

        def a_chunk_dma(sys, t):
            return pltpu.make_async_copy(
                a_hbm.at[pl.ds(chunk_row(sys, t), m_per), :],
                astage.at[sys],
                dma_sems.at[sys],
            )

        def dot16(sys):
            col0, w = COLS[sys]
            return jnp.dot(astage[sys, :, :].astype(jnp.bfloat16),
                           b16[:, col0:col0 + w],
                           preferred_element_type=jnp.float32)

        def step_rdma(sys, t, s, r):
            return pltpu.make_async_remote_copy(
                src_ref=comm.at[sys, s], dst_ref=comm.at[sys, r],
                send_sem=send_sems.at[sys, s],
                recv_sem=recv_sems.at[sys, r],
                device_id=(nbr(sys, t),),
                device_id_type=pl.DeviceIdType.MESH,
            )

        barrier_sem = pltpu.get_barrier_semaphore()
        for e in (1, 2, 4):
            pl.semaphore_signal(
                barrier_sem, inc=1,
                device_id=(to_vertex(v ^ e),),
                device_id_type=pl.DeviceIdType.MESH,
            )
        pl.semaphore_wait(barrier_sem, 3)

        for sys in range(N_SYS):
            a_chunk_dma(sys, 0).start()
        for sys in range(N_SYS):
            col0, w = COLS[sys]
            for off in range(0, w, 512):
                pw = min(512, w - off)
                bdma = pltpu.make_async_copy(
                    b_hbm.at[:, pl.ds(col0 + off, pw)],
                    bstage.at[:, pl.ds(0, pw)],
                    dma_sems.at[N_SYS])
                bdma.start()
                bdma.wait()
                b16[:, col0 + off:col0 + off + pw] = (
                    bstage[:, 0:pw].astype(jnp.bfloat16))
            a_chunk_dma(sys, 0).wait()
            comm[sys, 0, :, 0:w] = dot16(sys).astype(jnp.bfloat16)
            step_rdma(sys, 0, 0, 1).start()
        for sys in range(N_SYS):
            a_chunk_dma(sys, 1).start()

        for t in range(N_STEPS):
            s = t % 2
            r = (t + 1) % 2
            for sys in range(N_SYS):
                pltpu.make_async_copy(
                    a_hbm.at[pl.ds(0, m_per), :], astage.at[sys],
                    dma_sems.at[sys]).wait()
            for sys in range(N_SYS):
                col0, w = COLS[sys]
                pbuf[sys, :, 0:w] = dot16(sys)
            if t < N_STEPS - 1:
                for sys in range(N_SYS):
                    a_chunk_dma(sys, t + 2).start()
            for sys in range(N_SYS):
                step_rdma(sys, t, s, r).wait_send()
                if t <= N_STEPS - 2:
                    pl.semaphore_signal(
                        ready_sems.at[sys, s], inc=1,
                        device_id=(nbr(sys, t + 1),),
                        device_id_type=pl.DeviceIdType.MESH,
                    )
            for sys in range(N_SYS):
                col0, w = COLS[sys]
                step_rdma(sys, t, s, r).wait_recv()
                if t < N_STEPS - 1:
                    comm[sys, r, :, 0:w] = (
                        comm[sys, r, :, 0:w].astype(jnp.float32)
                        + pbuf[sys, :, 0:w]
                    ).astype(jnp.bfloat16)
                    pl.semaphore_wait(ready_sems.at[sys, s], 1)
                    step_rdma(sys, t + 1, r, s).start()
                else:
                    out_ref[:, col0:col0 + w] = (
                        comm[sys, r, :, 0:w].astype(jnp.float32)
                        + pbuf[sys, :, 0:w]
                    )

    return pl.pallas_call(
        body,
        out_shape=jax.ShapeDtypeStruct((m_per, n), jnp.float32),
        in_specs=[
            pl.BlockSpec(memory_space=pl.ANY),
            pl.BlockSpec(memory_space=pl.ANY),
        ],
        out_specs=pl.BlockSpec(memory_space=pltpu.VMEM),
        scratch_shapes=[
            pltpu.VMEM((k_per, n), jnp.bfloat16),
            pltpu.VMEM((k_per, 512), jnp.float32),
            pltpu.VMEM((N_SYS, m_per, k_per), jnp.float32),
            pltpu.VMEM((N_SYS, 2, m_per, NCMAX), jnp.bfloat16),
            pltpu.VMEM((N_SYS, m_per, NCMAX), jnp.float32),
            pltpu.SemaphoreType.DMA((N_SYS + 1,)),
            pltpu.SemaphoreType.DMA((N_SYS, 2)),
            pltpu.SemaphoreType.DMA((N_SYS, 2)),
            pltpu.SemaphoreType.REGULAR((N_SYS, 2)),
        ],
        compiler_params=pltpu.CompilerParams(
            collective_id=0,
            vmem_limit_bytes=64 * 1024 * 1024,
        ),
    )(A, B)
